# baseline (speedup 1.0000x reference)
import jax
import jax.numpy as jnp
from jax import lax
from jax.experimental import pallas as pl
from jax.experimental.pallas import tpu as pltpu
from jax.experimental.pallas import tpu_sc as plsc

VOCAB_TOTAL = 1000000
BATCH = 4096
SEQ = 200
D = 64
DPAD = 128
NC = 2
NS = 16
NW = NC * NS
WB = BATCH // NW
NBUF = 4
AHEAD = 2
GROUPS = SEQ // NBUF

CBLK = 128
NFULL = VOCAB_TOTAL // CBLK
VTAIL = NFULL * CBLK
TSTEPS = NFULL // NW + 1


def _cvt_body(tblt_hbm, tail_hbm, out_hbm, src_v, dst_v, tail_v):
    wid = lax.axis_index("s") * NC + lax.axis_index("c")

    def step(t, carry):
        j = wid + t * NW

        @pl.when(j < NFULL)
        def _():
            pltpu.sync_copy(tblt_hbm.at[:, pl.ds(j * CBLK, CBLK)], src_v)
            for d in range(D):
                for g in range(CBLK // 16):
                    v = src_v[d, pl.ds(g * 16, 16)]
                    plsc.store_scatter(
                        dst_v,
                        [g * 16 + lax.iota(jnp.int32, 16),
                         jnp.full((16,), d, jnp.int32)],
                        v)
            pltpu.sync_copy(dst_v, out_hbm.at[pl.ds(j * CBLK, CBLK)])

        return carry

    lax.fori_loop(0, TSTEPS, step, 0)

    @pl.when(wid == 0)
    def _():
        pltpu.sync_copy(tail_hbm, tail_v)
        pltpu.sync_copy(tail_v, out_hbm.at[pl.ds(VTAIL, VOCAB_TOTAL - VTAIL)])


@jax.jit
def _cvt_call(tblt, tail):
    mesh = plsc.VectorSubcoreMesh(core_axis_name="c", subcore_axis_name="s")
    return pl.kernel(
        _cvt_body,
        mesh=mesh,
        out_type=jax.ShapeDtypeStruct((VOCAB_TOTAL, DPAD), jnp.float32),
        scratch_types=[
            pltpu.VMEM((D, CBLK), jnp.float32),
            pltpu.VMEM((CBLK, DPAD), jnp.float32),
            pltpu.VMEM((VOCAB_TOTAL - VTAIL, DPAD), jnp.float32),
        ],
        compiler_params=pltpu.CompilerParams(
            use_tc_tiling_on_sc=True, needs_layout_passes=False),
    )(tblt, tail)


def _emb_body(xt_hbm, tbl_hbm, out_hbm, idx_v, rows_v, *sems):
    gsems, osems = sems[:NBUF], sems[NBUF:]
    wid = lax.axis_index("s") * NC + lax.axis_index("c")
    bbase = wid * WB
    pltpu.sync_copy(xt_hbm.at[:, pl.ds(bbase, WB)], idx_v)

    def fire_gather(s, slot):
        pltpu.async_copy(
            tbl_hbm.at[idx_v.at[s]],
            rows_v.at[pl.ds(slot * WB, WB)],
            gsems[slot])

    def wait_gather(slot):
        pltpu.make_async_copy(
            tbl_hbm.at[idx_v.at[0]],
            rows_v.at[pl.ds(slot * WB, WB)],
            gsems[slot]).wait()

    def fire_scatter(s, slot):
        pltpu.async_copy(
            rows_v.at[pl.ds(slot * WB, WB)],
            out_hbm.at[s, pl.ds(bbase, WB), :],
            osems[slot])

    def wait_scatter(slot):
        pltpu.make_async_copy(
            rows_v.at[pl.ds(slot * WB, WB)],
            out_hbm.at[0, pl.ds(bbase, WB), :],
            osems[slot]).wait()

    def process(s, slot, wait_scat, fire_ahead):
        wait_gather(slot)
        fire_scatter(s, slot)
        if fire_ahead:
            slot2 = (slot + AHEAD) % NBUF
            if wait_scat:
                wait_scatter(slot2)
            fire_gather(s + AHEAD, slot2)

    for s in range(AHEAD):
        fire_gather(s, s % NBUF)

    for b in range(NBUF):
        process(b, b, wait_scat=(b + AHEAD >= NBUF), fire_ahead=True)

    def group(m, carry):
        s0 = m * NBUF
        for b in range(NBUF):
            process(s0 + b, b, wait_scat=True, fire_ahead=True)
        return carry

    lax.fori_loop(1, GROUPS - 1, group, 0)

    s0 = (GROUPS - 1) * NBUF
    for b in range(NBUF):
        process(s0 + b, b, wait_scat=True, fire_ahead=(b + AHEAD < NBUF))
    for b in range(NBUF):
        wait_scatter(b)


@jax.jit
def _emb_call(xt, tbl):
    mesh = plsc.VectorSubcoreMesh(core_axis_name="c", subcore_axis_name="s")
    return pl.kernel(
        _emb_body,
        mesh=mesh,
        out_type=jax.ShapeDtypeStruct((SEQ, BATCH, DPAD), jnp.float32),
        scratch_types=[
            pltpu.VMEM((SEQ, WB), jnp.int32),
            pltpu.VMEM((NBUF * WB, DPAD), jnp.float32),
        ] + [pltpu.SemaphoreType.DMA] * (2 * NBUF),
        compiler_params=pltpu.CompilerParams(use_tc_tiling_on_sc=True),
    )(xt, tbl)


def kernel(x, table):
    xt = jnp.transpose(x)
    tblt = jnp.transpose(table)
    tail = jnp.pad(table[VTAIL:], ((0, 0), (0, DPAD - D)))
    tbl = _cvt_call(tblt, tail)
    return _emb_call(xt, tbl)[:, :, :D]


# speedup vs baseline: 1.4517x; 1.4517x over previous
import jax
import jax.numpy as jnp
from jax import lax
from jax.experimental import pallas as pl
from jax.experimental.pallas import tpu as pltpu
from jax.experimental.pallas import tpu_sc as plsc

VOCAB_TOTAL = 1000000
BATCH = 4096
SEQ = 200
D = 64
DPAD = 128
NC = 2
NS = 16
NW = NC * NS
WB = BATCH // NW
NBUF = 4
AHEAD = 2
GROUPS = SEQ // NBUF

def _emb_body(xt_hbm, tbl_hbm, out_hbm, idx_v, rows_v, *sems):
    gsems, osems = sems[:NBUF], sems[NBUF:]
    wid = lax.axis_index("s") * NC + lax.axis_index("c")
    bbase = wid * WB
    pltpu.sync_copy(xt_hbm.at[:, pl.ds(bbase, WB)], idx_v)

    def fire_gather(s, slot):
        pltpu.async_copy(
            tbl_hbm.at[idx_v.at[s]],
            rows_v.at[pl.ds(slot * WB, WB)],
            gsems[slot])

    def wait_gather(slot):
        pltpu.make_async_copy(
            tbl_hbm.at[idx_v.at[0]],
            rows_v.at[pl.ds(slot * WB, WB)],
            gsems[slot]).wait()

    def fire_scatter(s, slot):
        pltpu.async_copy(
            rows_v.at[pl.ds(slot * WB, WB)],
            out_hbm.at[s, pl.ds(bbase, WB), :],
            osems[slot])

    def wait_scatter(slot):
        pltpu.make_async_copy(
            rows_v.at[pl.ds(slot * WB, WB)],
            out_hbm.at[0, pl.ds(bbase, WB), :],
            osems[slot]).wait()

    def process(s, slot, wait_scat, fire_ahead):
        wait_gather(slot)
        fire_scatter(s, slot)
        if fire_ahead:
            slot2 = (slot + AHEAD) % NBUF
            if wait_scat:
                wait_scatter(slot2)
            fire_gather(s + AHEAD, slot2)

    for s in range(AHEAD):
        fire_gather(s, s % NBUF)

    for b in range(NBUF):
        process(b, b, wait_scat=(b + AHEAD >= NBUF), fire_ahead=True)

    def group(m, carry):
        s0 = m * NBUF
        for b in range(NBUF):
            process(s0 + b, b, wait_scat=True, fire_ahead=True)
        return carry

    lax.fori_loop(1, GROUPS - 1, group, 0)

    s0 = (GROUPS - 1) * NBUF
    for b in range(NBUF):
        process(s0 + b, b, wait_scat=True, fire_ahead=(b + AHEAD < NBUF))
    for b in range(NBUF):
        wait_scatter(b)


@jax.jit
def _emb_call(xt, tbl):
    mesh = plsc.VectorSubcoreMesh(core_axis_name="c", subcore_axis_name="s")
    return pl.kernel(
        _emb_body,
        mesh=mesh,
        out_type=jax.ShapeDtypeStruct((SEQ, BATCH, DPAD), jnp.float32),
        scratch_types=[
            pltpu.VMEM((SEQ, WB), jnp.int32),
            pltpu.VMEM((NBUF * WB, DPAD), jnp.float32),
        ] + [pltpu.SemaphoreType.DMA] * (2 * NBUF),
        compiler_params=pltpu.CompilerParams(use_tc_tiling_on_sc=True),
    )(xt, tbl)


def kernel(x, table):
    xt = jnp.transpose(x)
    tbl = jnp.reshape(table, (VOCAB_TOTAL // 2, DPAD))
    out = _emb_call(xt >> 1, tbl)
    odd = (xt & 1)[:, :, None] == 1
    return jnp.where(odd, out[:, :, D:], out[:, :, :D])


# speedup vs baseline: 2.3331x; 1.6071x over previous
import jax
import jax.numpy as jnp
from jax import lax
from jax.experimental import pallas as pl
from jax.experimental.pallas import tpu as pltpu
from jax.experimental.pallas import tpu_sc as plsc

VOCAB_TOTAL = 1000000
BATCH = 4096
SEQ = 200
D = 64
DPAD = 128
NC = 2
NS = 16
NW = NC * NS
WB = BATCH // NW
NBUF = 4
AHEAD = 2
GROUPS = SEQ // NBUF

def _emb_body(xt_hbm, tbl_hbm, out_hbm, idx_v, rows_v, *sems):
    gsems, osems = sems[:NBUF], sems[NBUF:]
    wid = lax.axis_index("s") * NC + lax.axis_index("c")
    bbase = wid * WB
    pltpu.sync_copy(xt_hbm.at[:, pl.ds(bbase, WB)], idx_v)

    def fire_gather(s, slot):
        pltpu.async_copy(
            tbl_hbm.at[idx_v.at[s]],
            rows_v.at[pl.ds(slot * WB, WB)],
            gsems[slot])

    def wait_gather(slot):
        pltpu.make_async_copy(
            tbl_hbm.at[idx_v.at[0]],
            rows_v.at[pl.ds(slot * WB, WB)],
            gsems[slot]).wait()

    def fire_scatter(s, slot):
        pltpu.async_copy(
            rows_v.at[pl.ds(slot * WB, WB)],
            out_hbm.at[s, pl.ds(bbase, WB), :],
            osems[slot])

    def wait_scatter(slot):
        pltpu.make_async_copy(
            rows_v.at[pl.ds(slot * WB, WB)],
            out_hbm.at[0, pl.ds(bbase, WB), :],
            osems[slot]).wait()

    def process(s, slot, wait_scat, fire_ahead):
        wait_gather(slot)
        fire_scatter(s, slot)
        if fire_ahead:
            slot2 = (slot + AHEAD) % NBUF
            if wait_scat:
                wait_scatter(slot2)
            fire_gather(s + AHEAD, slot2)

    for s in range(AHEAD):
        fire_gather(s, s % NBUF)

    for b in range(NBUF):
        process(b, b, wait_scat=(b + AHEAD >= NBUF), fire_ahead=True)

    def group(m, carry):
        s0 = m * NBUF
        for b in range(NBUF):
            process(s0 + b, b, wait_scat=True, fire_ahead=True)
        return carry

    lax.fori_loop(1, GROUPS - 1, group, 0)

    s0 = (GROUPS - 1) * NBUF
    for b in range(NBUF):
        process(s0 + b, b, wait_scat=True, fire_ahead=(b + AHEAD < NBUF))
    for b in range(NBUF):
        wait_scatter(b)


@jax.jit
def _emb_call(xt, tbl):
    mesh = plsc.VectorSubcoreMesh(core_axis_name="c", subcore_axis_name="s")
    return pl.kernel(
        _emb_body,
        mesh=mesh,
        out_type=jax.ShapeDtypeStruct((SEQ, BATCH, DPAD), jnp.float32),
        scratch_types=[
            pltpu.VMEM((SEQ, WB), jnp.int32),
            pltpu.VMEM((NBUF * WB, DPAD), jnp.float32),
        ] + [pltpu.SemaphoreType.DMA] * (2 * NBUF),
        compiler_params=pltpu.CompilerParams(use_tc_tiling_on_sc=True),
    )(xt, tbl)


def kernel(x, table):
    xt = jnp.transpose(x)
    tbl = jnp.pad(table, ((0, 0), (0, DPAD - D)))
    return _emb_call(xt, tbl)[:, :, :D]
